# Initial kernel scaffold; baseline (speedup 1.0000x reference)
import jax
import jax.numpy as jnp
from jax.experimental import pallas as pl


def kernel(x_num, x_cat, tables, W, b):
    raise NotImplementedError("write your pallas kernel here")


# speedup vs baseline: 68.4828x; 68.4828x over previous
import functools

import jax
import jax.numpy as jnp
from jax import lax
from jax.experimental import pallas as pl
from jax.experimental.pallas import tpu as pltpu
from jax.experimental.pallas import tpu_sc as plsc

B, FN, FC, V, D, NCLS = 16384, 13, 26, 16, 16, 2
NC, NS, LANES = 2, 16, 16
NW = NC * NS
CH = B // NW
NBLK = CH // LANES


def _sc_body(xcat_hbm, xnum_hbm, tabt_hbm, wemb_hbm, wnumb_hbm, biasb_hbm,
             out_hbm, xcat_v, xnum_v, tabt_v, wemb_v, wnumb_v, biasb_v,
             l_v, out_v, sem):
    cid = lax.axis_index("c")
    sid = lax.axis_index("s")
    wid = sid * NC + cid
    base = wid * CH

    cp_cat = pltpu.async_copy(xcat_hbm.at[pl.ds(base * FC, CH * FC)], xcat_v, sem)
    cp_num = pltpu.async_copy(xnum_hbm.at[pl.ds(base * FN, CH * FN)], xnum_v, sem)
    pltpu.sync_copy(tabt_hbm, tabt_v)
    pltpu.sync_copy(wemb_hbm, wemb_v)
    pltpu.sync_copy(wnumb_hbm, wnumb_v)
    pltpu.sync_copy(biasb_hbm, biasb_v)

    for i in range(FC):
        for c in range(NCLS):
            wvec = wemb_v[pl.ds((c * FC + i) * D, LANES)]
            acc = jnp.zeros((LANES,), jnp.float32)
            for d in range(D):
                acc = acc + tabt_v[pl.ds(i * (D * V) + d * V, LANES)] * wvec[d]
            l_v[pl.ds(c * (FC * V) + i * V, LANES)] = acc

    cp_cat.wait()
    cp_num.wait()

    iot = lax.iota(jnp.int32, LANES)

    def blk(j, carry):
        rows = j * LANES + iot
        acc0 = biasb_v[pl.ds(0, LANES)]
        acc1 = biasb_v[pl.ds(LANES, LANES)]
        roffn = rows * FN
        for n in range(FN):
            xv = plsc.load_gather(xnum_v, [roffn + n])
            acc0 = acc0 + xv * wnumb_v[pl.ds(n * LANES, LANES)]
            acc1 = acc1 + xv * wnumb_v[pl.ds((FN + n) * LANES, LANES)]
        roffc = rows * FC
        for i in range(FC):
            ci = plsc.load_gather(xcat_v, [roffc + i])
            acc0 = acc0 + plsc.load_gather(l_v, [ci + i * V])
            acc1 = acc1 + plsc.load_gather(l_v, [ci + (FC + i) * V])
        o2 = rows * NCLS
        plsc.store_scatter(out_v, [o2], acc0)
        plsc.store_scatter(out_v, [o2 + 1], acc1)
        return carry

    lax.fori_loop(0, NBLK, blk, 0)

    pltpu.sync_copy(out_v, out_hbm.at[pl.ds(base * NCLS, CH * NCLS)])


def kernel(x_num, x_cat, tables, W, b):
    x_cat_f = x_cat.astype(jnp.int32).reshape(-1)
    x_num_f = x_num.reshape(-1)
    tabt = tables.transpose(0, 2, 1).reshape(-1)
    wemb = W[:, FN:].reshape(-1)
    wnumb = jnp.broadcast_to(W[:, :FN][:, :, None],
                             (NCLS, FN, LANES)).reshape(-1)
    biasb = jnp.broadcast_to(b[:, None], (NCLS, LANES)).reshape(-1)

    mesh = plsc.VectorSubcoreMesh(core_axis_name="c", subcore_axis_name="s")
    run = functools.partial(
        pl.kernel,
        mesh=mesh,
        compiler_params=pltpu.CompilerParams(needs_layout_passes=False),
        out_type=jax.ShapeDtypeStruct((B * NCLS,), jnp.float32),
        scratch_types=[
            pltpu.VMEM((CH * FC,), jnp.int32),
            pltpu.VMEM((CH * FN,), jnp.float32),
            pltpu.VMEM((FC * D * V,), jnp.float32),
            pltpu.VMEM((NCLS * FC * D,), jnp.float32),
            pltpu.VMEM((NCLS * FN * LANES,), jnp.float32),
            pltpu.VMEM((NCLS * LANES,), jnp.float32),
            pltpu.VMEM((NCLS * FC * V,), jnp.float32),
            pltpu.VMEM((CH * NCLS,), jnp.float32),
            pltpu.SemaphoreType.DMA,
        ],
    )(_sc_body)
    out_flat = run(x_cat_f, x_num_f, tabt, wemb, wnumb, biasb)
    return out_flat.reshape(B, NCLS)


# speedup vs baseline: 72.3880x; 1.0570x over previous
import functools

import jax
import jax.numpy as jnp
from jax import lax
from jax.experimental import pallas as pl
from jax.experimental.pallas import tpu as pltpu
from jax.experimental.pallas import tpu_sc as plsc

B, FN, FC, V, D, NCLS = 16384, 13, 26, 16, 16, 2
NC, NS, LANES = 2, 16, 16
NW = NC * NS
CH = B // NW
NBLK = CH // LANES


_TAB_OFF = 0
_WEMB_OFF = _TAB_OFF + FC * D * V
_WNUM_OFF = _WEMB_OFF + NCLS * FC * D
_BIAS_OFF = _WNUM_OFF + NCLS * FN * LANES
CONST_LEN = _BIAS_OFF + NCLS * LANES


def _sc_body(consts_hbm, xcat_hbm, xnum_hbm,
             out_hbm, consts_v, xcat_v, xnum_v, l_v, out_v, sem):
    cid = lax.axis_index("c")
    sid = lax.axis_index("s")
    wid = sid * NC + cid
    base = wid * CH

    cp_con = pltpu.async_copy(consts_hbm, consts_v, sem)
    cp_cat = pltpu.async_copy(xcat_hbm.at[pl.ds(base * FC, CH * FC)], xcat_v, sem)
    cp_num = pltpu.async_copy(xnum_hbm.at[pl.ds(base * FN, CH * FN)], xnum_v, sem)
    cp_con.wait()

    for i in range(FC):
        for c in range(NCLS):
            wvec = consts_v[pl.ds(_WEMB_OFF + (c * FC + i) * D, LANES)]
            acc = jnp.zeros((LANES,), jnp.float32)
            for d in range(D):
                acc = acc + consts_v[pl.ds(_TAB_OFF + i * (D * V) + d * V,
                                           LANES)] * wvec[d]
            l_v[pl.ds(c * (FC * V) + i * V, LANES)] = acc

    cp_cat.wait()
    cp_num.wait()

    iot = lax.iota(jnp.int32, LANES)

    def blk(j, carry):
        rows = j * LANES + iot
        acc0 = consts_v[pl.ds(_BIAS_OFF, LANES)]
        acc1 = consts_v[pl.ds(_BIAS_OFF + LANES, LANES)]
        roffn = rows * FN
        for n in range(FN):
            xv = plsc.load_gather(xnum_v, [roffn + n])
            acc0 = acc0 + xv * consts_v[pl.ds(_WNUM_OFF + n * LANES, LANES)]
            acc1 = acc1 + xv * consts_v[pl.ds(_WNUM_OFF + (FN + n) * LANES,
                                              LANES)]
        roffc = rows * FC
        for i in range(FC):
            ci = plsc.load_gather(xcat_v, [roffc + i])
            acc0 = acc0 + plsc.load_gather(l_v, [ci + i * V])
            acc1 = acc1 + plsc.load_gather(l_v, [ci + (FC + i) * V])
        o2 = rows * NCLS
        plsc.store_scatter(out_v, [o2], acc0)
        plsc.store_scatter(out_v, [o2 + 1], acc1)
        return carry

    lax.fori_loop(0, NBLK, blk, 0)

    pltpu.sync_copy(out_v, out_hbm.at[pl.ds(base * NCLS, CH * NCLS)])


def kernel(x_num, x_cat, tables, W, b):
    x_cat_f = x_cat.astype(jnp.int32).reshape(-1)
    x_num_f = x_num.reshape(-1)
    tabt = tables.transpose(0, 2, 1).reshape(-1)
    wemb = W[:, FN:].reshape(-1)
    wnumb = jnp.broadcast_to(W[:, :FN][:, :, None],
                             (NCLS, FN, LANES)).reshape(-1)
    biasb = jnp.broadcast_to(b[:, None], (NCLS, LANES)).reshape(-1)
    consts = jnp.concatenate([tabt, wemb, wnumb, biasb])

    mesh = plsc.VectorSubcoreMesh(core_axis_name="c", subcore_axis_name="s")
    run = functools.partial(
        pl.kernel,
        mesh=mesh,
        compiler_params=pltpu.CompilerParams(needs_layout_passes=False,
                                             skip_device_barrier=True),
        out_type=jax.ShapeDtypeStruct((B * NCLS,), jnp.float32),
        scratch_types=[
            pltpu.VMEM((CONST_LEN,), jnp.float32),
            pltpu.VMEM((CH * FC,), jnp.int32),
            pltpu.VMEM((CH * FN,), jnp.float32),
            pltpu.VMEM((NCLS * FC * V,), jnp.float32),
            pltpu.VMEM((CH * NCLS,), jnp.float32),
            pltpu.SemaphoreType.DMA,
        ],
    )(_sc_body)
    out_flat = run(consts, x_cat_f, x_num_f)
    return out_flat.reshape(B, NCLS)


# speedup vs baseline: 93.3692x; 1.2898x over previous
import functools

import jax
import jax.numpy as jnp
from jax import lax
from jax.experimental import pallas as pl
from jax.experimental.pallas import tpu as pltpu
from jax.experimental.pallas import tpu_sc as plsc

B, FN, FC, V, D, NCLS = 16384, 13, 26, 16, 16, 2
NC, NS, LANES = 2, 16, 16
NW = NC * NS
CH = B // NW
CHK = 128
NCHK = CH // CHK
NBLK = CHK // LANES

_TAB_OFF = 0
_WEMB_OFF = _TAB_OFF + FC * D * V
_WNUM_OFF = _WEMB_OFF + NCLS * FC * D
_BIAS_OFF = _WNUM_OFF + NCLS * FN * LANES
CONST_LEN = _BIAS_OFF + NCLS * LANES


def _sc_body(consts_hbm, xcat_hbm, xnum_hbm, out_hbm,
             consts_v, l_v, xcat0, xcat1, xnum0, xnum1, out0, out1,
             csem, isem0, isem1, osem0, osem1):
    cid = lax.axis_index("c")
    sid = lax.axis_index("s")
    wid = sid * NC + cid
    base = wid * CH

    xcats = [xcat0, xcat1]
    xnums = [xnum0, xnum1]
    outs = [out0, out1]
    isems = [isem0, isem1]
    osems = [osem0, osem1]

    cp_con = pltpu.async_copy(consts_hbm, consts_v, csem)

    def start_in(k):
        s = k & 1
        r0 = base + k * CHK
        return (
            pltpu.async_copy(xcat_hbm.at[pl.ds(r0, CHK), :], xcats[s],
                             isems[s]),
            pltpu.async_copy(xnum_hbm.at[pl.ds(r0, CHK), :], xnums[s],
                             isems[s]),
        )

    pend = {0: start_in(0)}

    cp_con.wait()

    for i in range(FC):
        for c in range(NCLS):
            wvec = consts_v[pl.ds(_WEMB_OFF + (c * FC + i) * D, LANES)]
            acc = jnp.zeros((LANES,), jnp.float32)
            for d in range(D):
                acc = acc + consts_v[pl.ds(_TAB_OFF + i * (D * V) + d * V,
                                           LANES)] * wvec[d]
            l_v[pl.ds(c * (FC * V) + i * V, LANES)] = acc

    iot = lax.iota(jnp.int32, LANES)
    ow = {}
    for k in range(NCHK):
        s = k & 1
        if k + 1 < NCHK:
            pend[k + 1] = start_in(k + 1)
        for cp in pend.pop(k):
            cp.wait()
        if k >= 2:
            ow.pop(k - 2).wait()

        xcat_v, xnum_v, out_v = xcats[s], xnums[s], outs[s]

        def blk(j, carry):
            rows = j * LANES + iot
            acc0 = consts_v[pl.ds(_BIAS_OFF, LANES)]
            acc1 = consts_v[pl.ds(_BIAS_OFF + LANES, LANES)]
            for n in range(FN):
                col = jnp.full((LANES,), n, jnp.int32)
                xv = plsc.load_gather(xnum_v, [rows, col])
                acc0 = acc0 + xv * consts_v[pl.ds(_WNUM_OFF + n * LANES,
                                                  LANES)]
                acc1 = acc1 + xv * consts_v[pl.ds(_WNUM_OFF + (FN + n) * LANES,
                                                  LANES)]
            for i in range(FC):
                col = jnp.full((LANES,), i, jnp.int32)
                ci = plsc.load_gather(xcat_v, [rows, col])
                acc0 = acc0 + plsc.load_gather(l_v, [ci + i * V])
                acc1 = acc1 + plsc.load_gather(l_v, [ci + (FC + i) * V])
            zc = jnp.zeros((LANES,), jnp.int32)
            plsc.store_scatter(out_v, [rows, zc], acc0)
            plsc.store_scatter(out_v, [rows, zc + 1], acc1)
            return carry

        lax.fori_loop(0, NBLK, blk, 0)
        ow[k] = pltpu.async_copy(
            out_v, out_hbm.at[pl.ds(base + k * CHK, CHK), :], osems[s])

    for k in sorted(ow):
        ow.pop(k).wait()


def kernel(x_num, x_cat, tables, W, b):
    x_cat_i = x_cat.astype(jnp.int32)
    tabt = tables.transpose(0, 2, 1).reshape(-1)
    wemb = W[:, FN:].reshape(-1)
    wnumb = jnp.broadcast_to(W[:, :FN][:, :, None],
                             (NCLS, FN, LANES)).reshape(-1)
    biasb = jnp.broadcast_to(b[:, None], (NCLS, LANES)).reshape(-1)
    consts = jnp.concatenate([tabt, wemb, wnumb, biasb])

    mesh = plsc.VectorSubcoreMesh(core_axis_name="c", subcore_axis_name="s")
    run = functools.partial(
        pl.kernel,
        mesh=mesh,
        compiler_params=pltpu.CompilerParams(needs_layout_passes=False,
                                             skip_device_barrier=True,
                                             use_tc_tiling_on_sc=True),
        out_type=jax.ShapeDtypeStruct((B, NCLS), jnp.float32),
        scratch_types=[
            pltpu.VMEM((CONST_LEN,), jnp.float32),
            pltpu.VMEM((NCLS * FC * V,), jnp.float32),
            pltpu.VMEM((CHK, FC), jnp.int32),
            pltpu.VMEM((CHK, FC), jnp.int32),
            pltpu.VMEM((CHK, FN), jnp.float32),
            pltpu.VMEM((CHK, FN), jnp.float32),
            pltpu.VMEM((CHK, NCLS), jnp.float32),
            pltpu.VMEM((CHK, NCLS), jnp.float32),
            pltpu.SemaphoreType.DMA,
            pltpu.SemaphoreType.DMA,
            pltpu.SemaphoreType.DMA,
            pltpu.SemaphoreType.DMA,
            pltpu.SemaphoreType.DMA,
        ],
    )(_sc_body)
    return run(consts, x_cat_i, x_num)


# speedup vs baseline: 93.5564x; 1.0020x over previous
import functools

import jax
import jax.numpy as jnp
from jax import lax
from jax.experimental import pallas as pl
from jax.experimental.pallas import tpu as pltpu
from jax.experimental.pallas import tpu_sc as plsc

B, FN, FC, V, D, NCLS = 16384, 13, 26, 16, 16, 2
NC, NS, LANES = 2, 16, 16
NW = NC * NS
CH = B // NW
CHK = 128
NCHK = CH // CHK
NBLK = CHK // LANES

_TAB_OFF = 0
_WEMB_OFF = _TAB_OFF + FC * D * V
_WNUM_OFF = _WEMB_OFF + NCLS * FC * D
_BIAS_OFF = _WNUM_OFF + NCLS * FN * LANES
CONST_LEN = _BIAS_OFF + NCLS * LANES


def _sc_body(consts_hbm, xcat_hbm, xnum_hbm, out_hbm,
             consts_v, l_v, xcat0, xcat1, xnum0, xnum1, out0, out1,
             csem, isem0, isem1, osem0, osem1):
    cid = lax.axis_index("c")
    sid = lax.axis_index("s")
    wid = sid * NC + cid
    base = wid * CH

    xcats = [xcat0, xcat1]
    xnums = [xnum0, xnum1]
    outs = [out0, out1]
    isems = [isem0, isem1]
    osems = [osem0, osem1]

    cp_con = pltpu.async_copy(consts_hbm, consts_v, csem)

    def start_in(k):
        s = k & 1
        r0 = base + k * CHK
        return (
            pltpu.async_copy(xcat_hbm.at[pl.ds(r0, CHK), :], xcats[s],
                             isems[s]),
            pltpu.async_copy(xnum_hbm.at[pl.ds(r0, CHK), :], xnums[s],
                             isems[s]),
        )

    pend = {0: start_in(0)}

    cp_con.wait()

    with jax.named_scope("Lpre"):
        for i in range(FC):
            for c in range(NCLS):
                wvec = consts_v[pl.ds(_WEMB_OFF + (c * FC + i) * D, LANES)]
                acc = jnp.zeros((LANES,), jnp.float32)
                for d in range(D):
                    acc = acc + consts_v[pl.ds(_TAB_OFF + i * (D * V) + d * V,
                                               LANES)] * wvec[d]
                l_v[pl.ds(c * (FC * V) + i * V, LANES)] = acc

    iot = lax.iota(jnp.int32, LANES)
    ow = {}
    for k in range(NCHK):
        s = k & 1
        if k + 1 < NCHK:
            pend[k + 1] = start_in(k + 1)
        with jax.named_scope(f"wait_in{k}"):
            for cp in pend.pop(k):
                cp.wait()
            if k >= 2:
                ow.pop(k - 2).wait()

        xcat_v, xnum_v, out_v = xcats[s], xnums[s], outs[s]

        def blk(j, carry):
            rows = j * LANES + iot
            acc0 = consts_v[pl.ds(_BIAS_OFF, LANES)]
            acc1 = consts_v[pl.ds(_BIAS_OFF + LANES, LANES)]
            for n in range(FN):
                col = jnp.full((LANES,), n, jnp.int32)
                xv = plsc.load_gather(xnum_v, [rows, col])
                acc0 = acc0 + xv * consts_v[pl.ds(_WNUM_OFF + n * LANES,
                                                  LANES)]
                acc1 = acc1 + xv * consts_v[pl.ds(_WNUM_OFF + (FN + n) * LANES,
                                                  LANES)]
            for i in range(FC):
                col = jnp.full((LANES,), i, jnp.int32)
                ci = plsc.load_gather(xcat_v, [rows, col])
                acc0 = acc0 + plsc.load_gather(l_v, [ci + i * V])
                acc1 = acc1 + plsc.load_gather(l_v, [ci + (FC + i) * V])
            zc = jnp.zeros((LANES,), jnp.int32)
            plsc.store_scatter(out_v, [rows, zc], acc0)
            plsc.store_scatter(out_v, [rows, zc + 1], acc1)
            return carry

        with jax.named_scope(f"main{k}"):
            lax.fori_loop(0, NBLK, blk, 0)
        ow[k] = pltpu.async_copy(
            out_v, out_hbm.at[pl.ds(base + k * CHK, CHK), :], osems[s])

    with jax.named_scope("wait_out"):
        for k in sorted(ow):
            ow.pop(k).wait()


def kernel(x_num, x_cat, tables, W, b):
    x_cat_i = x_cat.astype(jnp.int32)
    tabt = tables.transpose(0, 2, 1).reshape(-1)
    wemb = W[:, FN:].reshape(-1)
    wnumb = jnp.broadcast_to(W[:, :FN][:, :, None],
                             (NCLS, FN, LANES)).reshape(-1)
    biasb = jnp.broadcast_to(b[:, None], (NCLS, LANES)).reshape(-1)
    consts = jnp.concatenate([tabt, wemb, wnumb, biasb])

    mesh = plsc.VectorSubcoreMesh(core_axis_name="c", subcore_axis_name="s")
    run = functools.partial(
        pl.kernel,
        mesh=mesh,
        compiler_params=pltpu.CompilerParams(needs_layout_passes=False,
                                             skip_device_barrier=True,
                                             use_tc_tiling_on_sc=True),
        out_type=jax.ShapeDtypeStruct((B, NCLS), jnp.float32),
        scratch_types=[
            pltpu.VMEM((CONST_LEN,), jnp.float32),
            pltpu.VMEM((NCLS * FC * V,), jnp.float32),
            pltpu.VMEM((CHK, FC), jnp.int32),
            pltpu.VMEM((CHK, FC), jnp.int32),
            pltpu.VMEM((CHK, FN), jnp.float32),
            pltpu.VMEM((CHK, FN), jnp.float32),
            pltpu.VMEM((CHK, NCLS), jnp.float32),
            pltpu.VMEM((CHK, NCLS), jnp.float32),
            pltpu.SemaphoreType.DMA,
            pltpu.SemaphoreType.DMA,
            pltpu.SemaphoreType.DMA,
            pltpu.SemaphoreType.DMA,
            pltpu.SemaphoreType.DMA,
        ],
    )(_sc_body)
    return run(consts, x_cat_i, x_num)
